# baseline (speedup 1.0000x reference)
import functools
import math

import jax
import jax.numpy as jnp
from jax import lax
from jax.experimental import pallas as pl
from jax.experimental.pallas import tpu as pltpu
from jax.experimental.pallas import tpu_sc as plsc

BATCH = 1024
SEQ = 200
EMB_DIM = 128

NC = 2
NS = 16
L = 16
NW = NC * NS
NTOK = BATCH * SEQ
BPW = NTOK // NW
CH = 128
NCHUNK = BPW // CH
DV = EMB_DIM // L


def _gelu_tc_body(x_ref, o_ref):
    x = x_ref[...]
    o_ref[...] = 0.5 * x * (1.0 + lax.erf(x * (1.0 / math.sqrt(2.0))))


def _gelu_table(mask_table):
    return pl.pallas_call(
        _gelu_tc_body,
        out_shape=jax.ShapeDtypeStruct(mask_table.shape, mask_table.dtype),
    )(mask_table)


NBUF = 4
SLOTS = 2


def _sc_body(sent_hbm, mask_hbm, char_hbm, g_hbm, out_hbm,
             idx_v, msk_v, g_v, r0, r1, r2, r3, spm,
             sem_g, sem_x, sem_s):
    rows = [r0, r1, r2, r3]
    cid = lax.axis_index("c")
    sid = lax.axis_index("s")
    wid = sid * NC + cid
    pltpu.sync_copy(sent_hbm.at[wid], idx_v)
    pltpu.sync_copy(mask_hbm.at[wid], msk_v)
    pltpu.sync_copy(g_hbm, g_v)
    g0 = [g_v[0, pl.ds(d * L, L)] for d in range(DV)]
    gd = [g_v[1, pl.ds(d * L, L)] - g0[d] for d in range(DV)]
    base_row = wid * BPW

    def gather(c, b):
        return pltpu.make_async_copy(
            char_hbm.at[idx_v.at[c]], rows[b], sem_g.at[b])

    def xcopy(b, s):
        return pltpu.make_async_copy(rows[b], spm.at[sid, s], sem_x.at[s])

    def store(c, s):
        return pltpu.make_async_copy(
            spm.at[sid, s], out_hbm.at[pl.ds(base_row + c * CH, CH)],
            sem_s.at[s])

    def compute(c, b):
        def grp_body(gi, tcarry):
            mrow = msk_v[c, pl.ds(gi * L, L)].astype(jnp.float32)
            for jj in range(L):
                j = gi * L + jj
                mf = lax.broadcast(mrow[jj], (L,))
                for d in range(DV):
                    row = rows[b][j, pl.ds(d * L, L)]
                    gate = g0[d] + mf * gd[d]
                    rows[b][j, pl.ds(d * L, L)] = row * gate
            return tcarry

        lax.fori_loop(0, CH // L, grp_body, 0)

    def step(c, b, s):
        nc = c + 2
        nb = (b + 2) % NBUF

        @pl.when(nc < NCHUNK)
        def _():
            gather(nc, nb).start()

        gather(c, b).wait()
        compute(c, b)

        @pl.when(c >= SLOTS)
        def _():
            store(c - SLOTS, s).wait()

        xcopy(b, s).start()
        xcopy(b, s).wait()
        store(c, s).start()

    gather(0, 0).start()
    gather(1, 1).start()

    def outer(i, carry):
        for j in range(NBUF):
            step(i * NBUF + j, j, j % SLOTS)
        return carry

    lax.fori_loop(0, (NCHUNK - 2) // NBUF, outer, 0)
    step(NCHUNK - 2, 0, 0)
    step(NCHUNK - 1, 1, 1)
    store(NCHUNK - 2, 0).wait()
    store(NCHUNK - 1, 1).wait()


@functools.partial(jax.jit, static_argnames=())
def kernel(sent, mask, char_table, mask_table):
    g = _gelu_table(mask_table)
    sent3 = sent.reshape(NW, NCHUNK, CH)
    mask3 = mask.reshape(NW, NCHUNK, CH)
    mesh = plsc.VectorSubcoreMesh(
        core_axis_name="c", subcore_axis_name="s",
        num_cores=NC, num_subcores=NS)
    out = pl.kernel(
        _sc_body,
        out_type=jax.ShapeDtypeStruct((NTOK, EMB_DIM), jnp.float32),
        mesh=mesh,
        scratch_types=[
            pltpu.VMEM((NCHUNK, CH), jnp.int32),
            pltpu.VMEM((NCHUNK, CH), jnp.int32),
            pltpu.VMEM((2, EMB_DIM), jnp.float32),
        ] + [pltpu.VMEM((CH, EMB_DIM), jnp.float32) for _ in range(NBUF)] + [
            pltpu.VMEM_SHARED((NS, SLOTS, CH, EMB_DIM), jnp.float32),
            pltpu.SemaphoreType.DMA((NBUF,)),
            pltpu.SemaphoreType.DMA((SLOTS,)),
            pltpu.SemaphoreType.DMA((SLOTS,)),
        ],
    )(sent3, mask3, char_table, g)
    return out.reshape(BATCH, SEQ, EMB_DIM)


# speedup vs baseline: 1.0200x; 1.0200x over previous
import functools
import math

import jax
import jax.numpy as jnp
from jax import lax
from jax.experimental import pallas as pl
from jax.experimental.pallas import tpu as pltpu
from jax.experimental.pallas import tpu_sc as plsc

BATCH = 1024
SEQ = 200
EMB_DIM = 128

NC = 2
NS = 16
L = 16
NW = NC * NS
NTOK = BATCH * SEQ
BPW = NTOK // NW
CH = 128
NCHUNK = BPW // CH
DV = EMB_DIM // L


def _gelu_tc_body(x_ref, o_ref):
    x = x_ref[...]
    o_ref[...] = 0.5 * x * (1.0 + lax.erf(x * (1.0 / math.sqrt(2.0))))


def _gelu_table(mask_table):
    return pl.pallas_call(
        _gelu_tc_body,
        out_shape=jax.ShapeDtypeStruct(mask_table.shape, mask_table.dtype),
    )(mask_table)


NBUF = 5


def _sc_body(sent_hbm, mask_hbm, char_hbm, g_hbm, out_hbm,
             idx_v, msk_v, g_v, r0, r1, r2, r3, r4, sem_g, sem_s):
    rows = [r0, r1, r2, r3, r4]
    cid = lax.axis_index("c")
    sid = lax.axis_index("s")
    wid = sid * NC + cid
    pltpu.sync_copy(sent_hbm.at[wid], idx_v)
    base_row = wid * BPW

    def gather(c, b):
        return pltpu.make_async_copy(
            char_hbm.at[idx_v.at[c]], rows[b], sem_g.at[b])

    def store(c, b):
        return pltpu.make_async_copy(
            rows[b], out_hbm.at[pl.ds(base_row + c * CH, CH)], sem_s.at[b])

    gather(0, 0).start()
    gather(1, 1).start()
    pltpu.sync_copy(mask_hbm.at[wid], msk_v)
    pltpu.sync_copy(g_hbm, g_v)
    g0 = [g_v[0, pl.ds(d * L, L)] for d in range(DV)]
    gd = [g_v[1, pl.ds(d * L, L)] - g0[d] for d in range(DV)]

    def compute(c, b):
        def grp_body(gi, tcarry):
            mrow = msk_v[c, pl.ds(gi * L, L)].astype(jnp.float32)
            for jj in range(L):
                j = gi * L + jj
                mf = lax.broadcast(mrow[jj], (L,))
                for d in range(DV):
                    row = rows[b][j, pl.ds(d * L, L)]
                    gate = g0[d] + mf * gd[d]
                    rows[b][j, pl.ds(d * L, L)] = row * gate
            return tcarry

        lax.fori_loop(0, CH // L, grp_body, 0)

    def outer(i, carry):
        for b in range(NBUF):
            c = i * NBUF + b
            nc = c + 2
            nb = (b + 2) % NBUF

            @pl.when(nc < NCHUNK)
            def _():
                @pl.when(c >= NBUF - 2)
                def _():
                    store(c - (NBUF - 2), nb).wait()
                gather(nc, nb).start()

            gather(c, b).wait()
            compute(c, b)
            store(c, b).start()
        return carry

    lax.fori_loop(0, NCHUNK // NBUF, outer, 0)
    for b in range(NBUF):
        store(NCHUNK - NBUF + b, b).wait()


@functools.partial(jax.jit, static_argnames=())
def kernel(sent, mask, char_table, mask_table):
    g = _gelu_table(mask_table)
    sent3 = sent.reshape(NW, NCHUNK, CH)
    mask3 = mask.reshape(NW, NCHUNK, CH)
    mesh = plsc.VectorSubcoreMesh(
        core_axis_name="c", subcore_axis_name="s",
        num_cores=NC, num_subcores=NS)
    out = pl.kernel(
        _sc_body,
        out_type=jax.ShapeDtypeStruct((NTOK, EMB_DIM), jnp.float32),
        mesh=mesh,
        scratch_types=[
            pltpu.VMEM((NCHUNK, CH), jnp.int32),
            pltpu.VMEM((NCHUNK, CH), jnp.int32),
            pltpu.VMEM((2, EMB_DIM), jnp.float32),
        ] + [pltpu.VMEM((CH, EMB_DIM), jnp.float32) for _ in range(NBUF)] + [
            pltpu.SemaphoreType.DMA((NBUF,)),
            pltpu.SemaphoreType.DMA((NBUF,)),
        ],
    )(sent3, mask3, char_table, g)
    return out.reshape(BATCH, SEQ, EMB_DIM)
